# baseline (speedup 1.0000x reference)
import functools

import jax
import jax.numpy as jnp
from jax import lax
from jax.experimental import pallas as pl
from jax.experimental.pallas import tpu as pltpu
from jax.experimental.pallas import tpu_sc as plsc

_NC = 2
_NS = 16
_NW = _NC * _NS

_D = 64
_DP = 128
_L = 16
_CHUNK = 128


def _make_lookup(B):
    assert B % (_NW * _CHUNK) == 0
    per_w = B // _NW
    nch = per_w // _CHUNK
    assert nch % 2 == 0 and nch >= 4
    mesh = plsc.VectorSubcoreMesh(core_axis_name="c", subcore_axis_name="s")

    @functools.partial(
        pl.kernel,
        out_type=jax.ShapeDtypeStruct((B, _D), jnp.float32),
        mesh=mesh,
        scratch_types=[
            pltpu.VMEM((nch, _CHUNK), jnp.int32),
            pltpu.VMEM((_CHUNK, _DP), jnp.float32),
            pltpu.VMEM((_CHUNK, _DP), jnp.float32),
            pltpu.VMEM((_CHUNK, _D), jnp.float32),
            pltpu.VMEM((_CHUNK, _D), jnp.float32),
            pltpu.SemaphoreType.DMA,
            pltpu.SemaphoreType.DMA,
        ],
    )
    def lookup(table_hbm, idx_hbm, out_hbm, idx_v, g0, g1, p0, p1, sem0, sem1):
        wid = lax.axis_index("s") * _NC + lax.axis_index("c")
        base = pl.multiple_of(wid * per_w, _CHUNK)
        pltpu.sync_copy(idx_hbm.at[wid], idx_v)

        gbufs = (g0, g1)
        pbufs = (p0, p1)
        sems = (sem0, sem1)

        def fire(t, b):
            pltpu.async_copy(
                table_hbm.at[idx_v.at[t]], gbufs[b], sems[b]
            )

        def drain(b):
            pltpu.make_async_copy(
                table_hbm.at[idx_v.at[0]], gbufs[b], sems[b]
            ).wait()

        def repack(b):
            gb, pb = gbufs[b], pbufs[b]

            def blk(i, _):
                r0 = i * 8
                for j in range(8):
                    for k in range(_D // _L):
                        pb[r0 + j, pl.ds(k * _L, _L)] = gb[
                            r0 + j, pl.ds(k * _L, _L)
                        ]
                return 0

            lax.fori_loop(0, _CHUNK // 8, blk, 0)

        def copy_out(t, b):
            off = pl.multiple_of(base + t * _CHUNK, _CHUNK)
            pltpu.sync_copy(pbufs[b], out_hbm.at[pl.ds(off, _CHUNK)])

        fire(0, 0)

        def body(i0, _):
            t0 = 2 * i0
            fire(t0 + 1, 1)
            drain(0)
            repack(0)
            copy_out(t0, 0)
            fire(t0 + 2, 0)
            drain(1)
            repack(1)
            copy_out(t0 + 1, 1)
            return 0

        lax.fori_loop(0, (nch - 2) // 2, body, 0)

        fire(nch - 1, 1)
        drain(0)
        repack(0)
        copy_out(nch - 2, 0)
        drain(1)
        repack(1)
        copy_out(nch - 1, 1)

    return lookup


def kernel(sen_ids, table):
    S, T = sen_ids.shape
    B = S * T
    table_p = lax.pad(table, jnp.float32(0), ((0, 0, 0), (0, _DP - _D, 0)))
    idx = sen_ids.reshape(-1).astype(jnp.int32)
    idx3 = idx.reshape(_NW, B // (_NW * _CHUNK), _CHUNK)
    out = _make_lookup(B)(table_p, idx3)
    return out.reshape(S, T, _D)
